# Initial kernel scaffold; baseline (speedup 1.0000x reference)
import jax
import jax.numpy as jnp
from jax.experimental import pallas as pl


def kernel(features_of_nodes, node_ids_for_edges, features_of_edges, global_features, node_type_ids, edge_type_ids, batch_ids, edge_type_emb, node_type_emb, We1, be1, We2, be2, Wn1, bn1, Wn2, bn2, Wn3, bn3, Wn4, bn4, Wg1, bg1, Wg2, bg2):
    raise NotImplementedError("write your pallas kernel here")


# speedup vs baseline: 1.8472x; 1.8472x over previous
import functools

import jax
import jax.numpy as jnp
from jax import lax
from jax.experimental import pallas as pl
from jax.experimental.pallas import tpu as pltpu
from jax.experimental.pallas import tpu_sc as plsc

N = 10000
E = 160000
F = 256
FE = 16
B = 16
FU = 64
NT = 4
ET = 8
TE = 16
H = 256

CHUNK = 128
E_PAD = 163840
NCHUNK = E_PAD // CHUNK
NWORK = 32
GCH_PER_W = NCHUNK // NWORK
SCH_PER_T = NCHUNK // 16
BN = 1000
BE = 1024


def _prec_body(x_ref, ohb_ref, wcat_ref, u_ref, au_ref, te_ref, ate_ref,
               ntm_ref, bt_ref, bu_ref, be1_ref, bn1_ref,
               trow_ref, tcol_ref, tabs_ref):
    p = x_ref[...] @ wcat_ref[...]
    u_tab = u_ref[...] @ au_ref[...]
    t0 = p[:, :H] + ohb_ref[...] @ u_tab + be1_ref[...]
    t2 = p[:, H:2 * H] + bn1_ref[...]
    trow_ref[...] = jnp.concatenate([t0, t2], axis=1)
    tcol_ref[...] = p[:, 2 * H:]

    @pl.when(pl.program_id(0) == 0)
    def _():
        te_tab = te_ref[...] @ ate_ref[...]
        nt_tab = ntm_ref[...] @ bt_ref[...]
        ub_tab = u_ref[...] @ bu_ref[...]
        tabs_ref[...] = jnp.concatenate(
            [te_tab, nt_tab, jnp.zeros((4, H), jnp.float32), ub_tab], axis=0)


def _precompute(x, ohb, wcat, u, a_u, te_emb, a_te, nt_emb, b_t, b_u, be1, bn1):
    grid = N // BN
    full = lambda shape: pl.BlockSpec(shape, lambda i: (0, 0))
    return pl.pallas_call(
        _prec_body,
        grid=(grid,),
        in_specs=[
            pl.BlockSpec((BN, F), lambda i: (i, 0)),
            pl.BlockSpec((BN, B), lambda i: (i, 0)),
            full((F, 3 * H)), full((B, FU)), full((FU, H)),
            full((ET, TE)), full((TE, H)),
            full((NT, TE)), full((TE, H)), full((FU, H)),
            full((1, H)), full((1, H)),
        ],
        out_specs=[
            pl.BlockSpec((BN, 2 * H), lambda i: (i, 0)),
            pl.BlockSpec((BN, H), lambda i: (i, 0)),
            pl.BlockSpec((2 * B, H), lambda i: (0, 0)),
        ],
        out_shape=[
            jax.ShapeDtypeStruct((N, 2 * H), jnp.float32),
            jax.ShapeDtypeStruct((N, H), jnp.float32),
            jax.ShapeDtypeStruct((2 * B, H), jnp.float32),
        ],
    )(x, ohb, wcat, u, a_u, te_emb, a_te, nt_emb, b_t, b_u, be1, bn1)


def _gather_body(trow_hbm, tcol_hbm, ridx_hbm, cidx_hbm,
                 grow_hbm, gcol_hbm, ridx_v, cidx_v, rbuf, cbuf, sem_r, sem_c):
    wid = lax.axis_index("s") * 2 + lax.axis_index("c")
    c0 = wid * GCH_PER_W
    pltpu.sync_copy(ridx_hbm.at[pl.ds(c0, GCH_PER_W)], ridx_v)
    pltpu.sync_copy(cidx_hbm.at[pl.ds(c0, GCH_PER_W)], cidx_v)

    def body(j, carry):
        e0 = (c0 + j) * CHUNK
        cp_r = pltpu.async_copy(trow_hbm.at[ridx_v.at[j]], rbuf, sem_r)
        cp_c = pltpu.async_copy(tcol_hbm.at[cidx_v.at[j]], cbuf, sem_c)
        cp_r.wait()
        cp_c.wait()
        pltpu.sync_copy(rbuf, grow_hbm.at[pl.ds(e0, CHUNK)])
        pltpu.sync_copy(cbuf, gcol_hbm.at[pl.ds(e0, CHUNK)])
        return carry

    lax.fori_loop(0, GCH_PER_W, body, 0)


def _gather(trow, tcol, ridx, cidx):
    mesh = plsc.VectorSubcoreMesh(core_axis_name="c", subcore_axis_name="s")
    fn = functools.partial(
        pl.kernel, _gather_body, mesh=mesh,
        out_type=[
            jax.ShapeDtypeStruct((E_PAD, 2 * H), jnp.float32),
            jax.ShapeDtypeStruct((E_PAD, H), jnp.float32),
        ],
        scratch_types=[
            pltpu.VMEM((GCH_PER_W, CHUNK), jnp.int32),
            pltpu.VMEM((GCH_PER_W, CHUNK), jnp.int32),
            pltpu.VMEM((CHUNK, 2 * H), jnp.float32),
            pltpu.VMEM((CHUNK, H), jnp.float32),
            pltpu.SemaphoreType.DMA,
            pltpu.SemaphoreType.DMA,
        ],
    )()
    return fn(trow, tcol, ridx, cidx)


def _edge_body(grow_ref, gcol_ref, ohet_ref, fe_ref, tabs_ref, afe_ref,
               we2_ref, be2_ref, me_ref, newe_ref, hm0_ref, hm1_ref):
    g = grow_ref[...]
    s = g[:, :H] + gcol_ref[...]
    h_e = jnp.maximum(
        s + fe_ref[...] @ afe_ref[...] + ohet_ref[...] @ tabs_ref[0:ET], 0.0)
    ne = h_e @ we2_ref[...] + be2_ref[...]
    newe_ref[...] = ne
    hm = jnp.maximum(g[:, H:] + ne @ me_ref[...], 0.0)
    rows = pl.program_id(0) * BE + lax.broadcasted_iota(jnp.int32, (BE, 1), 0)
    hm = jnp.where(rows < E, hm, 0.0)
    hm0_ref[...] = hm[:, :128]
    hm1_ref[...] = hm[:, 128:]


def _edge(grow, gcol, ohet, fe_pad, tabs, a_fe, we2, be2, m_e):
    grid = E_PAD // BE
    full = lambda shape: pl.BlockSpec(shape, lambda i: (0, 0))
    return pl.pallas_call(
        _edge_body,
        grid=(grid,),
        in_specs=[
            pl.BlockSpec((BE, 2 * H), lambda i: (i, 0)),
            pl.BlockSpec((BE, H), lambda i: (i, 0)),
            pl.BlockSpec((BE, ET), lambda i: (i, 0)),
            pl.BlockSpec((BE, FE), lambda i: (i, 0)),
            full((2 * B, H)), full((FE, H)), full((H, FE)), full((1, FE)),
            full((FE, H)),
        ],
        out_specs=[
            pl.BlockSpec((BE, FE), lambda i: (i, 0)),
            pl.BlockSpec((BE, 128), lambda i: (i, 0)),
            pl.BlockSpec((BE, 128), lambda i: (i, 0)),
        ],
        out_shape=[
            jax.ShapeDtypeStruct((E_PAD, FE), jnp.float32),
            jax.ShapeDtypeStruct((E_PAD, 128), jnp.float32),
            jax.ShapeDtypeStruct((E_PAD, 128), jnp.float32),
        ],
    )(grow, gcol, ohet, fe_pad, tabs, a_fe, we2, be2, m_e)


def _scatter_half(hm_ref, cidx_hbm, eones_hbm, idx_v, valbuf, onesbuf,
                  agg_sh, cnt_sh, s, with_cnt):
    def outer(b, carry):
        blk0 = s * SCH_PER_T + b * 8
        pltpu.sync_copy(cidx_hbm.at[pl.ds(blk0, 8)], idx_v)

        def inner(jj, carry2):
            e0 = (blk0 + jj) * CHUNK
            pltpu.sync_copy(hm_ref.at[pl.ds(e0, CHUNK)], valbuf)
            pltpu.sync_copy(valbuf, agg_sh.at[idx_v.at[jj]], add=True)
            if with_cnt:
                pltpu.sync_copy(eones_hbm.at[pl.ds(e0, CHUNK)], onesbuf)
                pltpu.sync_copy(onesbuf, cnt_sh.at[idx_v.at[jj]], add=True)
            return carry2

        lax.fori_loop(0, 8, inner, 0)
        return carry

    lax.fori_loop(0, SCH_PER_T // 8, outer, 0)


def _scatter_body(hm0_hbm, hm1_hbm, cidx_hbm, eones_hbm, z128_hbm, z16_hbm,
                  agg0_hbm, agg1_hbm, cnt_hbm,
                  idx_v, valbuf, onesbuf, agg_sh, cnt_sh):
    c = lax.axis_index("c")
    s = lax.axis_index("s")

    @pl.when(s == 0)
    def _():
        pltpu.sync_copy(z128_hbm, agg_sh)

        @pl.when(c == 0)
        def _():
            pltpu.sync_copy(z16_hbm, cnt_sh)

    plsc.subcore_barrier()

    @pl.when(c == 0)
    def _():
        _scatter_half(hm0_hbm, cidx_hbm, eones_hbm, idx_v, valbuf, onesbuf,
                      agg_sh, cnt_sh, s, True)

    @pl.when(c == 1)
    def _():
        _scatter_half(hm1_hbm, cidx_hbm, eones_hbm, idx_v, valbuf, onesbuf,
                      agg_sh, cnt_sh, s, False)

    plsc.subcore_barrier()
    n0 = s * 640

    @pl.when(s < 15)
    def _():
        @pl.when(c == 0)
        def _():
            pltpu.sync_copy(agg_sh.at[pl.ds(n0, 640)], agg0_hbm.at[pl.ds(n0, 640)])
            pltpu.sync_copy(cnt_sh.at[pl.ds(n0, 640)], cnt_hbm.at[pl.ds(n0, 640)])

        @pl.when(c == 1)
        def _():
            pltpu.sync_copy(agg_sh.at[pl.ds(n0, 640)], agg1_hbm.at[pl.ds(n0, 640)])

    @pl.when(s == 15)
    def _():
        @pl.when(c == 0)
        def _():
            pltpu.sync_copy(agg_sh.at[pl.ds(n0, 400)], agg0_hbm.at[pl.ds(n0, 400)])
            pltpu.sync_copy(cnt_sh.at[pl.ds(n0, 400)], cnt_hbm.at[pl.ds(n0, 400)])

        @pl.when(c == 1)
        def _():
            pltpu.sync_copy(agg_sh.at[pl.ds(n0, 400)], agg1_hbm.at[pl.ds(n0, 400)])


def _scatter(hm0, hm1, cidx, eones, z128, z16):
    mesh = plsc.VectorSubcoreMesh(core_axis_name="c", subcore_axis_name="s")
    fn = functools.partial(
        pl.kernel, _scatter_body, mesh=mesh,
        out_type=[
            jax.ShapeDtypeStruct((N, 128), jnp.float32),
            jax.ShapeDtypeStruct((N, 128), jnp.float32),
            jax.ShapeDtypeStruct((N, 16), jnp.float32),
        ],
        scratch_types=[
            pltpu.VMEM((8, CHUNK), jnp.int32),
            pltpu.VMEM((CHUNK, 128), jnp.float32),
            pltpu.VMEM((CHUNK, 16), jnp.float32),
            pltpu.VMEM_SHARED((N, 128), jnp.float32),
            pltpu.VMEM_SHARED((N, 16), jnp.float32),
        ],
    )()
    return fn(hm0, hm1, cidx, eones, z128, z16)


def _node_body(x_ref, agg0_ref, agg1_ref, cnt_ref, ohnt_ref, ohb_ref,
               tabs_ref, wn2_ref, bn2_ref, bx_ref, bm_ref, bn3_ref,
               wn4_ref, bn4_ref, u_ref, wg1_ref, bg1_ref, wg2_ref, bg2_ref,
               newx_ref, newu_ref, gsum_acc, gcnt_acc):
    i = pl.program_id(0)
    cnt = cnt_ref[...][:, 0:1]
    wn2 = wn2_ref[...]
    aggw = agg0_ref[...] @ wn2[:128] + agg1_ref[...] @ wn2[128:]
    mean = (aggw + cnt * bn2_ref[...]) / jnp.maximum(cnt, 1.0)
    tabs = tabs_ref[...]
    h = jnp.maximum(
        x_ref[...] @ bx_ref[...] + mean @ bm_ref[...]
        + ohnt_ref[...] @ tabs[ET:2 * ET] + ohb_ref[...] @ tabs[B:2 * B]
        + bn3_ref[...], 0.0)
    nx = h @ wn4_ref[...] + bn4_ref[...]
    newx_ref[...] = nx

    ohb = ohb_ref[...]
    part = lax.dot_general(ohb, nx, (((0,), (0,)), ((), ())))
    pcnt = lax.dot_general(ohb, jnp.ones((BN, 1), jnp.float32),
                           (((0,), (0,)), ((), ())))

    @pl.when(i == 0)
    def _():
        gsum_acc[...] = part
        gcnt_acc[...] = pcnt

    @pl.when(i > 0)
    def _():
        gsum_acc[...] += part
        gcnt_acc[...] += pcnt

    @pl.when(i == pl.num_programs(0) - 1)
    def _():
        gmean = gsum_acc[...] / jnp.maximum(gcnt_acc[...], 1.0)
        wg1 = wg1_ref[...]
        g1 = jnp.maximum(
            u_ref[...] @ wg1[:FU] + gmean @ wg1[FU:] + bg1_ref[...], 0.0)
        newu_ref[...] = g1 @ wg2_ref[...] + bg2_ref[...]


def _node(x, agg0, agg1, cnt, ohnt, ohb, tabs, wn2, bn2, b_x, b_m, bn3,
          wn4, bn4, u, wg1, bg1, wg2, bg2):
    grid = N // BN
    full = lambda shape: pl.BlockSpec(shape, lambda i: (0, 0))
    return pl.pallas_call(
        _node_body,
        grid=(grid,),
        in_specs=[
            pl.BlockSpec((BN, F), lambda i: (i, 0)),
            pl.BlockSpec((BN, 128), lambda i: (i, 0)),
            pl.BlockSpec((BN, 128), lambda i: (i, 0)),
            pl.BlockSpec((BN, 16), lambda i: (i, 0)),
            pl.BlockSpec((BN, ET), lambda i: (i, 0)),
            pl.BlockSpec((BN, B), lambda i: (i, 0)),
            full((2 * B, H)), full((H, H)), full((1, H)),
            full((F, H)), full((H, H)), full((1, H)),
            full((H, F)), full((1, F)),
            full((B, FU)), full((FU + F, 128)), full((1, 128)),
            full((128, FU)), full((1, FU)),
        ],
        out_specs=[
            pl.BlockSpec((BN, F), lambda i: (i, 0)),
            pl.BlockSpec((B, FU), lambda i: (0, 0)),
        ],
        out_shape=[
            jax.ShapeDtypeStruct((N, F), jnp.float32),
            jax.ShapeDtypeStruct((B, FU), jnp.float32),
        ],
        scratch_shapes=[
            pltpu.VMEM((B, F), jnp.float32),
            pltpu.VMEM((B, 1), jnp.float32),
        ],
    )(x, agg0, agg1, cnt, ohnt, ohb, tabs, wn2, bn2, b_x, b_m, bn3,
      wn4, bn4, u, wg1, bg1, wg2, bg2)


def kernel(features_of_nodes, node_ids_for_edges, features_of_edges,
           global_features, node_type_ids, edge_type_ids, batch_ids,
           edge_type_emb, node_type_emb, We1, be1, We2, be2, Wn1, bn1,
           Wn2, bn2, Wn3, bn3, Wn4, bn4, Wg1, bg1, Wg2, bg2):
    x = features_of_nodes
    row = node_ids_for_edges[0]
    col = node_ids_for_edges[1]

    a_s, a_d = We1[:F], We1[F:2 * F]
    a_fe = We1[2 * F:2 * F + FE]
    a_te = We1[2 * F + FE:2 * F + FE + TE]
    a_u = We1[2 * F + FE + TE:]
    m_x, m_e = Wn1[:F], Wn1[F:]
    b_x, b_m = Wn3[:F], Wn3[F:F + H]
    b_t, b_u = Wn3[F + H:F + H + TE], Wn3[F + H + TE:]
    wcat = jnp.concatenate([a_s, m_x, a_d], axis=1)

    ohb = (batch_ids[:, None] == jnp.arange(B)[None, :]).astype(jnp.float32)
    ohnt = (node_type_ids[:, None] == jnp.arange(ET)[None, :]).astype(jnp.float32)
    pad = E_PAD - E
    ohet = jnp.pad(
        (edge_type_ids[:, None] == jnp.arange(ET)[None, :]).astype(jnp.float32),
        ((0, pad), (0, 0)))
    fe_pad = jnp.pad(features_of_edges, ((0, pad), (0, 0)))
    ridx = jnp.pad(row, (0, pad)).reshape(NCHUNK, CHUNK)
    cidx = jnp.pad(col, (0, pad)).reshape(NCHUNK, CHUNK)
    eones = jnp.broadcast_to(
        (jnp.arange(E_PAD) < E).astype(jnp.float32)[:, None], (E_PAD, 16))
    z128 = jnp.zeros((N, 128), jnp.float32)
    z16 = jnp.zeros((N, 16), jnp.float32)

    trow, tcol, tabs = _precompute(
        x, ohb, wcat, global_features, a_u, edge_type_emb, a_te,
        node_type_emb, b_t, b_u, be1.reshape(1, H), bn1.reshape(1, H))

    grow, gcol = _gather(trow, tcol, ridx, cidx)

    new_e_pad, hm0, hm1 = _edge(
        grow, gcol, ohet, fe_pad, tabs, a_fe, We2, be2.reshape(1, FE), m_e)

    _c = cidx.reshape(-1)
    agg0 = jax.ops.segment_sum(hm0, _c, num_segments=N)
    agg1 = jax.ops.segment_sum(hm1, _c, num_segments=N)
    cnt = jax.ops.segment_sum(eones, _c, num_segments=N)

    new_x, new_u = _node(
        x, agg0, agg1, cnt, ohnt, ohb, tabs, Wn2, bn2.reshape(1, H),
        b_x, b_m, bn3.reshape(1, H), Wn4, bn4.reshape(1, F),
        global_features, Wg1, bg1.reshape(1, 128), Wg2, bg2.reshape(1, FU))

    return (new_x, new_e_pad[:E], new_u)


# speedup vs baseline: 2.5136x; 1.3607x over previous
import functools

import jax
import jax.numpy as jnp
from jax import lax
from jax.experimental import pallas as pl
from jax.experimental.pallas import tpu as pltpu
from jax.experimental.pallas import tpu_sc as plsc

N = 10000
E = 160000
F = 256
FE = 16
B = 16
FU = 64
NT = 4
ET = 8
TE = 16
H = 256

CHUNK = 128
E_PAD = 163840
NCHUNK = E_PAD // CHUNK
NWORK = 32
GCH_PER_W = NCHUNK // NWORK
SCH_PER_T = NCHUNK // 16
BN = 1000
BE = 1024


def _prec_body(x_ref, ohb_ref, wcat_ref, u_ref, au_ref, te_ref, ate_ref,
               ntm_ref, bt_ref, bu_ref, be1_ref, bn1_ref,
               trow_ref, tcol_ref, tabs_ref):
    p = x_ref[...] @ wcat_ref[...]
    u_tab = u_ref[...] @ au_ref[...]
    t0 = p[:, :H] + ohb_ref[...] @ u_tab + be1_ref[...]
    t2 = p[:, H:2 * H] + bn1_ref[...]
    trow_ref[...] = jnp.concatenate([t0, t2], axis=1)
    tcol_ref[...] = p[:, 2 * H:]

    @pl.when(pl.program_id(0) == 0)
    def _():
        te_tab = te_ref[...] @ ate_ref[...]
        nt_tab = ntm_ref[...] @ bt_ref[...]
        ub_tab = u_ref[...] @ bu_ref[...]
        tabs_ref[...] = jnp.concatenate(
            [te_tab, nt_tab, jnp.zeros((4, H), jnp.float32), ub_tab], axis=0)


def _precompute(x, ohb, wcat, u, a_u, te_emb, a_te, nt_emb, b_t, b_u, be1, bn1):
    grid = N // BN
    full = lambda shape: pl.BlockSpec(shape, lambda i: (0, 0))
    return pl.pallas_call(
        _prec_body,
        grid=(grid,),
        in_specs=[
            pl.BlockSpec((BN, F), lambda i: (i, 0)),
            pl.BlockSpec((BN, B), lambda i: (i, 0)),
            full((F, 3 * H)), full((B, FU)), full((FU, H)),
            full((ET, TE)), full((TE, H)),
            full((NT, TE)), full((TE, H)), full((FU, H)),
            full((1, H)), full((1, H)),
        ],
        out_specs=[
            pl.BlockSpec((BN, 2 * H), lambda i: (i, 0)),
            pl.BlockSpec((BN, H), lambda i: (i, 0)),
            pl.BlockSpec((2 * B, H), lambda i: (0, 0)),
        ],
        out_shape=[
            jax.ShapeDtypeStruct((N, 2 * H), jnp.float32),
            jax.ShapeDtypeStruct((N, H), jnp.float32),
            jax.ShapeDtypeStruct((2 * B, H), jnp.float32),
        ],
    )(x, ohb, wcat, u, a_u, te_emb, a_te, nt_emb, b_t, b_u, be1, bn1)


def _gather_body(trow_hbm, tcol_hbm, ridx_hbm, cidx_hbm,
                 grow_hbm, gcol_hbm, ridx_v, cidx_v, rbuf, cbuf, sem_r, sem_c):
    wid = lax.axis_index("s") * 2 + lax.axis_index("c")
    c0 = wid * GCH_PER_W
    pltpu.sync_copy(ridx_hbm.at[pl.ds(c0, GCH_PER_W)], ridx_v)
    pltpu.sync_copy(cidx_hbm.at[pl.ds(c0, GCH_PER_W)], cidx_v)

    def body(j, carry):
        e0 = (c0 + j) * CHUNK
        cp_r = pltpu.async_copy(trow_hbm.at[ridx_v.at[j]], rbuf, sem_r)
        cp_c = pltpu.async_copy(tcol_hbm.at[cidx_v.at[j]], cbuf, sem_c)
        cp_r.wait()
        cp_c.wait()
        pltpu.sync_copy(rbuf, grow_hbm.at[pl.ds(e0, CHUNK)])
        pltpu.sync_copy(cbuf, gcol_hbm.at[pl.ds(e0, CHUNK)])
        return carry

    lax.fori_loop(0, GCH_PER_W, body, 0)


def _gather(trow, tcol, ridx, cidx):
    mesh = plsc.VectorSubcoreMesh(core_axis_name="c", subcore_axis_name="s")
    fn = functools.partial(
        pl.kernel, _gather_body, mesh=mesh,
        out_type=[
            jax.ShapeDtypeStruct((E_PAD, 2 * H), jnp.float32),
            jax.ShapeDtypeStruct((E_PAD, H), jnp.float32),
        ],
        scratch_types=[
            pltpu.VMEM((GCH_PER_W, CHUNK), jnp.int32),
            pltpu.VMEM((GCH_PER_W, CHUNK), jnp.int32),
            pltpu.VMEM((CHUNK, 2 * H), jnp.float32),
            pltpu.VMEM((CHUNK, H), jnp.float32),
            pltpu.SemaphoreType.DMA,
            pltpu.SemaphoreType.DMA,
        ],
    )()
    return fn(trow, tcol, ridx, cidx)


def _edge_body(grow_ref, gcol_ref, ohet_ref, fe_ref, tabs_ref, afe_ref,
               we2_ref, be2_ref, me_ref, newe_ref, hm0_ref, hm1_ref):
    g = grow_ref[...]
    s = g[:, :H] + gcol_ref[...]
    h_e = jnp.maximum(
        s + fe_ref[...] @ afe_ref[...] + ohet_ref[...] @ tabs_ref[0:ET], 0.0)
    ne = h_e @ we2_ref[...] + be2_ref[...]
    newe_ref[...] = ne
    hm = jnp.maximum(g[:, H:] + ne @ me_ref[...], 0.0)
    rows = pl.program_id(0) * BE + lax.broadcasted_iota(jnp.int32, (BE, 1), 0)
    hm = jnp.where(rows < E, hm, 0.0)
    hm0_ref[...] = hm[:, :128]
    hm1_ref[...] = hm[:, 128:]


def _edge(grow, gcol, ohet, fe_pad, tabs, a_fe, we2, be2, m_e):
    grid = E_PAD // BE
    full = lambda shape: pl.BlockSpec(shape, lambda i: (0, 0))
    return pl.pallas_call(
        _edge_body,
        grid=(grid,),
        in_specs=[
            pl.BlockSpec((BE, 2 * H), lambda i: (i, 0)),
            pl.BlockSpec((BE, H), lambda i: (i, 0)),
            pl.BlockSpec((BE, ET), lambda i: (i, 0)),
            pl.BlockSpec((BE, FE), lambda i: (i, 0)),
            full((2 * B, H)), full((FE, H)), full((H, FE)), full((1, FE)),
            full((FE, H)),
        ],
        out_specs=[
            pl.BlockSpec((BE, FE), lambda i: (i, 0)),
            pl.BlockSpec((BE, 128), lambda i: (i, 0)),
            pl.BlockSpec((BE, 128), lambda i: (i, 0)),
        ],
        out_shape=[
            jax.ShapeDtypeStruct((E_PAD, FE), jnp.float32),
            jax.ShapeDtypeStruct((E_PAD, 128), jnp.float32),
            jax.ShapeDtypeStruct((E_PAD, 128), jnp.float32),
        ],
    )(grow, gcol, ohet, fe_pad, tabs, a_fe, we2, be2, m_e)


def _scatter_half(hm_ref, cidx_hbm, eones_hbm, idx_v, valbuf, onesbuf,
                  agg_sh, cnt_sh, s, with_cnt):
    def body(j, carry):
        e0 = (s * SCH_PER_T + j) * CHUNK
        pltpu.sync_copy(cidx_hbm.at[pl.ds(e0, CHUNK)], idx_v)
        pltpu.sync_copy(hm_ref.at[pl.ds(e0, CHUNK)], valbuf)
        pltpu.sync_copy(valbuf, agg_sh.at[idx_v], add=True)
        if with_cnt:
            pltpu.sync_copy(eones_hbm.at[pl.ds(e0, CHUNK)], onesbuf)
            pltpu.sync_copy(onesbuf, cnt_sh.at[idx_v], add=True)
        return carry

    lax.fori_loop(0, SCH_PER_T, body, 0)


def _scatter_body(hm0_hbm, hm1_hbm, cidx_hbm, eones_hbm, z128_hbm, z16_hbm,
                  agg0_hbm, agg1_hbm,
                  idx_v, valbuf, onesbuf, agg_sh):
    cnt_sh = None
    c = lax.axis_index("c")
    s = lax.axis_index("s")
    n0 = s * 640

    @pl.when(s < 15)
    def _():
        pltpu.sync_copy(z128_hbm.at[pl.ds(n0, 640)], agg_sh.at[pl.ds(n0, 640)])

    @pl.when(s == 15)
    def _():
        pltpu.sync_copy(z128_hbm.at[pl.ds(n0, 400)], agg_sh.at[pl.ds(n0, 400)])

    plsc.subcore_barrier()

    @pl.when(c == 0)
    def _():
        _scatter_half(hm0_hbm, cidx_hbm, eones_hbm, idx_v, valbuf, onesbuf,
                      agg_sh, cnt_sh, s, False)

    @pl.when(c == 1)
    def _():
        _scatter_half(hm1_hbm, cidx_hbm, eones_hbm, idx_v, valbuf, onesbuf,
                      agg_sh, cnt_sh, s, False)

    plsc.subcore_barrier()

    @pl.when(s < 15)
    def _():
        @pl.when(c == 0)
        def _():
            pltpu.sync_copy(agg_sh.at[pl.ds(n0, 640)], agg0_hbm.at[pl.ds(n0, 640)])

        @pl.when(c == 1)
        def _():
            pltpu.sync_copy(agg_sh.at[pl.ds(n0, 640)], agg1_hbm.at[pl.ds(n0, 640)])

    @pl.when(s == 15)
    def _():
        @pl.when(c == 0)
        def _():
            pltpu.sync_copy(agg_sh.at[pl.ds(n0, 400)], agg0_hbm.at[pl.ds(n0, 400)])

        @pl.when(c == 1)
        def _():
            pltpu.sync_copy(agg_sh.at[pl.ds(n0, 400)], agg1_hbm.at[pl.ds(n0, 400)])


def _scatter(hm0, hm1, cidx, eones, z128, z16):
    mesh = plsc.VectorSubcoreMesh(core_axis_name="c", subcore_axis_name="s")
    fn = functools.partial(
        pl.kernel, _scatter_body, mesh=mesh,
        out_type=[
            jax.ShapeDtypeStruct((N, 128), jnp.float32),
            jax.ShapeDtypeStruct((N, 128), jnp.float32),
        ],
        scratch_types=[
            pltpu.VMEM((CHUNK,), jnp.int32),
            pltpu.VMEM((CHUNK, 128), jnp.float32),
            pltpu.VMEM((CHUNK, 16), jnp.float32),
            pltpu.VMEM_SHARED((N, 128), jnp.float32),
        ],
    )()
    return fn(hm0, hm1, cidx, eones, z128, z16)


def _node_body(x_ref, agg0_ref, agg1_ref, cnt_ref, ohnt_ref, ohb_ref,
               tabs_ref, wn2_ref, bn2_ref, bx_ref, bm_ref, bn3_ref,
               wn4_ref, bn4_ref, u_ref, wg1_ref, bg1_ref, wg2_ref, bg2_ref,
               newx_ref, newu_ref, gsum_acc, gcnt_acc):
    i = pl.program_id(0)
    cnt = cnt_ref[...][:, 0:1]
    wn2 = wn2_ref[...]
    aggw = agg0_ref[...] @ wn2[:128] + agg1_ref[...] @ wn2[128:]
    mean = (aggw + cnt * bn2_ref[...]) / jnp.maximum(cnt, 1.0)
    tabs = tabs_ref[...]
    h = jnp.maximum(
        x_ref[...] @ bx_ref[...] + mean @ bm_ref[...]
        + ohnt_ref[...] @ tabs[ET:2 * ET] + ohb_ref[...] @ tabs[B:2 * B]
        + bn3_ref[...], 0.0)
    nx = h @ wn4_ref[...] + bn4_ref[...]
    newx_ref[...] = nx

    ohb = ohb_ref[...]
    part = lax.dot_general(ohb, nx, (((0,), (0,)), ((), ())))
    pcnt = lax.dot_general(ohb, jnp.ones((BN, 1), jnp.float32),
                           (((0,), (0,)), ((), ())))

    @pl.when(i == 0)
    def _():
        gsum_acc[...] = part
        gcnt_acc[...] = pcnt

    @pl.when(i > 0)
    def _():
        gsum_acc[...] += part
        gcnt_acc[...] += pcnt

    @pl.when(i == pl.num_programs(0) - 1)
    def _():
        gmean = gsum_acc[...] / jnp.maximum(gcnt_acc[...], 1.0)
        wg1 = wg1_ref[...]
        g1 = jnp.maximum(
            u_ref[...] @ wg1[:FU] + gmean @ wg1[FU:] + bg1_ref[...], 0.0)
        newu_ref[...] = g1 @ wg2_ref[...] + bg2_ref[...]


def _node(x, agg0, agg1, cnt, ohnt, ohb, tabs, wn2, bn2, b_x, b_m, bn3,
          wn4, bn4, u, wg1, bg1, wg2, bg2):
    grid = N // BN
    full = lambda shape: pl.BlockSpec(shape, lambda i: (0, 0))
    return pl.pallas_call(
        _node_body,
        grid=(grid,),
        in_specs=[
            pl.BlockSpec((BN, F), lambda i: (i, 0)),
            pl.BlockSpec((BN, 128), lambda i: (i, 0)),
            pl.BlockSpec((BN, 128), lambda i: (i, 0)),
            pl.BlockSpec((BN, 16), lambda i: (i, 0)),
            pl.BlockSpec((BN, ET), lambda i: (i, 0)),
            pl.BlockSpec((BN, B), lambda i: (i, 0)),
            full((2 * B, H)), full((H, H)), full((1, H)),
            full((F, H)), full((H, H)), full((1, H)),
            full((H, F)), full((1, F)),
            full((B, FU)), full((FU + F, 128)), full((1, 128)),
            full((128, FU)), full((1, FU)),
        ],
        out_specs=[
            pl.BlockSpec((BN, F), lambda i: (i, 0)),
            pl.BlockSpec((B, FU), lambda i: (0, 0)),
        ],
        out_shape=[
            jax.ShapeDtypeStruct((N, F), jnp.float32),
            jax.ShapeDtypeStruct((B, FU), jnp.float32),
        ],
        scratch_shapes=[
            pltpu.VMEM((B, F), jnp.float32),
            pltpu.VMEM((B, 1), jnp.float32),
        ],
    )(x, agg0, agg1, cnt, ohnt, ohb, tabs, wn2, bn2, b_x, b_m, bn3,
      wn4, bn4, u, wg1, bg1, wg2, bg2)


def kernel(features_of_nodes, node_ids_for_edges, features_of_edges,
           global_features, node_type_ids, edge_type_ids, batch_ids,
           edge_type_emb, node_type_emb, We1, be1, We2, be2, Wn1, bn1,
           Wn2, bn2, Wn3, bn3, Wn4, bn4, Wg1, bg1, Wg2, bg2):
    x = features_of_nodes
    row = node_ids_for_edges[0]
    col = node_ids_for_edges[1]

    a_s, a_d = We1[:F], We1[F:2 * F]
    a_fe = We1[2 * F:2 * F + FE]
    a_te = We1[2 * F + FE:2 * F + FE + TE]
    a_u = We1[2 * F + FE + TE:]
    m_x, m_e = Wn1[:F], Wn1[F:]
    b_x, b_m = Wn3[:F], Wn3[F:F + H]
    b_t, b_u = Wn3[F + H:F + H + TE], Wn3[F + H + TE:]
    wcat = jnp.concatenate([a_s, m_x, a_d], axis=1)

    ohb = (batch_ids[:, None] == jnp.arange(B)[None, :]).astype(jnp.float32)
    ohnt = (node_type_ids[:, None] == jnp.arange(ET)[None, :]).astype(jnp.float32)
    pad = E_PAD - E
    ohet = jnp.pad(
        (edge_type_ids[:, None] == jnp.arange(ET)[None, :]).astype(jnp.float32),
        ((0, pad), (0, 0)))
    fe_pad = jnp.pad(features_of_edges, ((0, pad), (0, 0)))
    cidx_flat = jnp.pad(col, (0, pad))
    ridx = jnp.pad(row, (0, pad)).reshape(NCHUNK, CHUNK)
    cidx = cidx_flat.reshape(NCHUNK, CHUNK)
    eones = jnp.broadcast_to(
        (jnp.arange(E_PAD) < E).astype(jnp.float32)[:, None], (E_PAD, 16))
    z128 = jnp.zeros((N, 128), jnp.float32)
    z16 = jnp.zeros((N, 16), jnp.float32)

    trow, tcol, tabs = _precompute(
        x, ohb, wcat, global_features, a_u, edge_type_emb, a_te,
        node_type_emb, b_t, b_u, be1.reshape(1, H), bn1.reshape(1, H))

    grow, gcol = _gather(trow, tcol, ridx, cidx)

    new_e_pad, hm0, hm1 = _edge(
        grow, gcol, ohet, fe_pad, tabs, a_fe, We2, be2.reshape(1, FE), m_e)

    agg0, agg1 = _scatter(hm0, hm1, cidx_flat, eones, z128, z16)
    cnt = jax.ops.segment_sum(eones, cidx_flat, num_segments=N)

    new_x, new_u = _node(
        x, agg0, agg1, cnt, ohnt, ohb, tabs, Wn2, bn2.reshape(1, H),
        b_x, b_m, bn3.reshape(1, H), Wn4, bn4.reshape(1, F),
        global_features, Wg1, bg1.reshape(1, 128), Wg2, bg2.reshape(1, FU))

    return (new_x, new_e_pad[:E], new_u)
